# Initial kernel scaffold; baseline (speedup 1.0000x reference)
import jax
import jax.numpy as jnp
from jax.experimental import pallas as pl


def kernel(state_bytes, sparse, byte_embed, block_pos, local_pos, l0_n1, l0_wq, l0_wk, l0_wv, l0_wo, l0_n2, l0_wg, l0_wu, l0_wd, l1_n1, l1_wq, l1_wk, l1_wv, l1_wo, l1_n2, l1_wg, l1_wu, l1_wd, final_norm, w_out):
    raise NotImplementedError("write your pallas kernel here")


# speedup vs baseline: 13.8984x; 13.8984x over previous
import functools

import jax
import jax.numpy as jnp
from jax import lax
from jax.experimental import pallas as pl
from jax.experimental.pallas import tpu as pltpu

DIM = 512
NUM_HEADS = 128
TOP_K = 32
BLOCK = 256
NUM_BLOCKS = 64
STATE = 16384
HIDDEN = int(DIM * 8 / 3)
EPS = 1.1920929e-07

QKV_T = 256
ATT_T = 128


def _rms(x, w):
    return x * lax.rsqrt(jnp.mean(x * x, axis=-1, keepdims=True) + EPS) * w


def _bf(x):
    return x.astype(jnp.bfloat16).astype(jnp.float32)


def _bdot(a, b):
    return jnp.dot(a.astype(jnp.bfloat16), b.astype(jnp.bfloat16),
                   preferred_element_type=jnp.float32)


def _attn_ffn_body(x_ref, q_ref, tk_ref, bk_ref, tv_ref, bv_ref,
                   wo_ref, n2_ref, wg_ref, wu_ref, wd_ref, o_ref):
    q = q_ref[...]
    qpos = q > 0.0
    tk = tk_ref[...]
    bk = bk_ref[...]
    q3 = q[:, None, :]
    qpos3 = qpos[:, None, :]
    ksel = jnp.where(qpos3, tk[None], bk[None])
    s3 = q3 * ksel
    sm = q * jnp.where(qpos, tk[TOP_K - 1][None, :], bk[0][None, :])
    p3 = jnp.exp(s3 - sm[:, None, :])
    ssum = p3.sum(axis=1, keepdims=True)
    a3 = _bf(p3 / ssum)
    tv = _bf(tv_ref[...])
    bv = _bf(bv_ref[...])
    outs = []
    for dd in range(4):
        vsel = jnp.where(qpos3, tv[None, :, dd, :], bv[None, :, dd, :])
        outs.append((a3 * vsel).sum(axis=1))
    attn = jnp.stack(outs, axis=-1).reshape(q.shape[0], DIM)
    x = x_ref[...] + _bdot(attn, wo_ref[...])
    xn = _rms(x, n2_ref[...])
    g = _bdot(xn, wg_ref[...])
    u = _bdot(xn, wu_ref[...])
    h = g * jax.nn.sigmoid(g) * u
    o_ref[...] = x + _bdot(h, wd_ref[...])


def _attn_ffn(x, q, tk, bk, tv, bv, woT, n2, wgT, wuT, wdT):
    T = ATT_T
    grid = (STATE // T,)
    full = lambda s: pl.BlockSpec(s, lambda i: (0,) * len(s))
    return pl.pallas_call(
        _attn_ffn_body,
        grid=grid,
        in_specs=[
            pl.BlockSpec((T, DIM), lambda i: (i, 0)),
            pl.BlockSpec((T, NUM_HEADS), lambda i: (i, 0)),
            full((TOP_K, NUM_HEADS)),
            full((TOP_K, NUM_HEADS)),
            full((TOP_K, 4, NUM_HEADS)),
            full((TOP_K, 4, NUM_HEADS)),
            full((DIM, DIM)),
            full((DIM,)),
            full((DIM, HIDDEN)),
            full((DIM, HIDDEN)),
            full((HIDDEN, DIM)),
        ],
        out_specs=pl.BlockSpec((T, DIM), lambda i: (i, 0)),
        out_shape=jax.ShapeDtypeStruct((STATE, DIM), jnp.float32),
    )(x, q, tk, bk, tv, bv, woT, n2, wgT, wuT, wdT)


def _final_body(x_ref, n_ref, w_ref, o_ref):
    xn = _rms(x_ref[...], n_ref[...])
    o_ref[...] = _bdot(xn, w_ref[...])


def _final(x, fn, woutT):
    T = ATT_T
    full = lambda s: pl.BlockSpec(s, lambda i: (0,) * len(s))
    return pl.pallas_call(
        _final_body,
        grid=(STATE // T,),
        in_specs=[
            pl.BlockSpec((T, DIM), lambda i: (i, 0)),
            full((DIM,)),
            full((DIM, 256)),
        ],
        out_specs=pl.BlockSpec((T, 256), lambda i: (i, 0)),
        out_shape=jax.ShapeDtypeStruct((STATE, 256), jnp.float32),
    )(x, fn, woutT)


def _layer(x, n1, wq, wk, wv, woT, n2, wgT, wuT, wdT):
    xn = _rms(x, n1)
    q_all = jnp.einsum('bsd,hd->bhs', xn, wq)
    keys = jnp.einsum('bsd,hd->bhs', xn, wk)
    b, S, d = x.shape
    dv = d // NUM_HEADS
    v = (xn @ wv.T).reshape(b, S, NUM_HEADS, dv).transpose(0, 2, 1, 3)
    k_idx = jnp.argsort(keys, axis=-1)
    topi = k_idx[:, :, -TOP_K:]
    boti = k_idx[:, :, :TOP_K]
    top_keys = jnp.take_along_axis(keys, topi, axis=-1)
    bot_keys = jnp.take_along_axis(keys, boti, axis=-1)
    top_vals = jnp.take_along_axis(v, topi[..., None], axis=2)
    bot_vals = jnp.take_along_axis(v, boti[..., None], axis=2)
    tk = top_keys[0].T
    bk = bot_keys[0].T
    tv = top_vals[0].transpose(1, 2, 0)
    bv = bot_vals[0].transpose(1, 2, 0)
    return _attn_ffn(x[0], q_all[0].T, tk, bk, tv, bv,
                     woT, n2, wgT, wuT, wdT)[None]


def kernel(state_bytes, sparse, byte_embed, block_pos, local_pos, l0_n1, l0_wq, l0_wk, l0_wv, l0_wo, l0_n2, l0_wg, l0_wu, l0_wd, l1_n1, l1_wq, l1_wk, l1_wv, l1_wo, l1_n2, l1_wg, l1_wu, l1_wd, final_norm, w_out):
    x = byte_embed[state_bytes]
    bp = jnp.repeat(block_pos, BLOCK, axis=0)[None]
    lp = jnp.tile(local_pos, (NUM_BLOCKS, 1))[None]
    x = x + bp + lp
    x = _layer(x, l0_n1, l0_wq, l0_wk, l0_wv, l0_wo.T, l0_n2,
               l0_wg.T, l0_wu.T, l0_wd.T)
    x = _layer(x, l1_n1, l1_wq, l1_wk, l1_wv, l1_wo.T, l1_n2,
               l1_wg.T, l1_wu.T, l1_wd.T)
    out = _final(x[0], final_norm, w_out.T)
    return out.reshape(1, STATE, 256)
